# Initial kernel scaffold; baseline (speedup 1.0000x reference)
import jax
import jax.numpy as jnp
from jax.experimental import pallas as pl


def kernel(x, edge_index, W1, b1, ln1_w, ln1_b, W2, b2, ln2_w, ln2_b):
    raise NotImplementedError("write your pallas kernel here")


# speedup vs baseline: 7.9554x; 7.9554x over previous
import functools

import jax
import jax.numpy as jnp
from jax import lax
from jax.experimental import pallas as pl
from jax.experimental.pallas import tpu as pltpu
from jax.experimental.pallas import tpu_sc as plsc

N = 10000
D = 128
E = 320000
EPS = 1e-5

NC = 2
NS = 16
NW = NC * NS
C = 128
NCHUNK = 80
EPW = NCHUNK * C
E_PAD = NW * EPW
ACC_ROWS = 10240
RPT = ACC_ROWS // NS
DEGW = 16

_mesh = plsc.VectorSubcoreMesh(core_axis_name="c", subcore_axis_name="s")


@functools.partial(
    pl.kernel,
    mesh=_mesh,
    out_type=jax.ShapeDtypeStruct((NC, ACC_ROWS, D), jnp.float32),
    scratch_types=[
        pltpu.VMEM((NCHUNK, C), jnp.int32),
        pltpu.VMEM((C, D), jnp.float32),
        pltpu.VMEM_SHARED((ACC_ROWS, D), jnp.float32),
    ],
)
def _deg_kernel(dsts_hbm, out_hbm, didx, buf, acc):
    cid = lax.axis_index("c")
    sid = lax.axis_index("s")
    wid = cid * NS + sid

    def _fill(val):
        def body(i, _):
            for k in range(D // 16):
                buf[i, pl.ds(k * 16, 16)] = jnp.full((16,), val, jnp.float32)
            return 0
        lax.fori_loop(0, C, body, 0)

    _fill(0.0)
    for t in range(RPT // C):
        pltpu.sync_copy(buf, acc.at[pl.ds(sid * RPT + t * C, C)])
    _fill(1.0)
    pltpu.sync_copy(dsts_hbm.at[wid], didx)
    plsc.subcore_barrier()

    def step(j, _):
        pltpu.sync_copy(buf, acc.at[didx.at[j]], add=True)
        return 0

    lax.fori_loop(0, NCHUNK, step, 0)
    plsc.subcore_barrier()
    pltpu.sync_copy(acc.at[pl.ds(sid * RPT, RPT)],
                    out_hbm.at[cid].at[pl.ds(sid * RPT, RPT)])


@functools.partial(
    pl.kernel,
    mesh=_mesh,
    out_type=jax.ShapeDtypeStruct((NC, ACC_ROWS, D), jnp.float32),
    scratch_types=[
        pltpu.VMEM((NCHUNK, C), jnp.int32),
        pltpu.VMEM((NCHUNK, C), jnp.int32),
        pltpu.VMEM((C, D), jnp.float32),
        pltpu.VMEM_SHARED((ACC_ROWS, D), jnp.float32),
    ],
)
def _scatter_kernel(g_hbm, srcs_hbm, dsts_hbm, out_hbm, sidx, didx, buf, acc):
    cid = lax.axis_index("c")
    sid = lax.axis_index("s")
    wid = cid * NS + sid

    def zero(i, _):
        for k in range(D // 16):
            buf[i, pl.ds(k * 16, 16)] = jnp.zeros((16,), jnp.float32)
        return 0

    lax.fori_loop(0, C, zero, 0)
    for t in range(RPT // C):
        pltpu.sync_copy(buf, acc.at[pl.ds(sid * RPT + t * C, C)])
    pltpu.sync_copy(srcs_hbm.at[wid], sidx)
    pltpu.sync_copy(dsts_hbm.at[wid], didx)
    plsc.subcore_barrier()

    def step(j, _):
        pltpu.sync_copy(g_hbm.at[sidx.at[j]], buf)
        pltpu.sync_copy(buf, acc.at[didx.at[j]], add=True)
        return 0

    lax.fori_loop(0, NCHUNK, step, 0)
    plsc.subcore_barrier()
    pltpu.sync_copy(acc.at[pl.ds(sid * RPT, RPT)],
                    out_hbm.at[cid].at[pl.ds(sid * RPT, RPT)])


BR = 400


def _dinv_from(deg_ref):
    deg = deg_ref[0] + deg_ref[1] + 1.0
    return deg, lax.rsqrt(deg)


def _pre_body(x_ref, w_ref, deg_ref, h_ref, g_ref):
    h = jnp.dot(x_ref[...], w_ref[...], preferred_element_type=jnp.float32)
    _, dinv = _dinv_from(deg_ref)
    h_ref[...] = h
    g_ref[...] = h * dinv


def _ln_relu(t, lnw, lnb):
    mu = jnp.mean(t, axis=1, keepdims=True)
    var = jnp.mean((t - mu) ** 2, axis=1, keepdims=True)
    return jnp.maximum((t - mu) * lax.rsqrt(var + EPS) * lnw + lnb, 0.0)


def _mid_body(acc_ref, h_ref, deg_ref, b_ref, lnw_ref, lnb_ref, w2_ref,
              h2_ref, g2_ref):
    deg, dinv = _dinv_from(deg_ref)
    t = (acc_ref[0] + acc_ref[1]) * dinv + h_ref[...] / deg + b_ref[...]
    y = _ln_relu(t, lnw_ref[...], lnb_ref[...])
    h2 = jnp.dot(y, w2_ref[...], preferred_element_type=jnp.float32)
    h2_ref[...] = h2
    g2_ref[...] = h2 * dinv


def _post_body(acc_ref, h_ref, deg_ref, b_ref, lnw_ref, lnb_ref, y_ref):
    deg, dinv = _dinv_from(deg_ref)
    t = (acc_ref[0] + acc_ref[1]) * dinv + h_ref[...] / deg + b_ref[...]
    y_ref[...] = _ln_relu(t, lnw_ref[...], lnb_ref[...])


_row_spec = pl.BlockSpec((BR, D), lambda i: (i, 0))
_mat_spec = pl.BlockSpec((D, D), lambda i: (0, 0))
_vec_spec = pl.BlockSpec((1, D), lambda i: (0, 0))
_deg_spec = pl.BlockSpec((NC, BR, D), lambda i: (0, i, 0))
_acc_spec = pl.BlockSpec((NC, BR, D), lambda i: (0, i, 0))
_f32 = jnp.float32


def _pre(x, W, deg2):
    return pl.pallas_call(
        _pre_body,
        grid=(N // BR,),
        in_specs=[_row_spec, _mat_spec, _deg_spec],
        out_specs=[_row_spec, _row_spec],
        out_shape=[jax.ShapeDtypeStruct((N, D), _f32)] * 2,
    )(x, W, deg2)


def _mid(acc2, h, deg2, b, lnw, lnb, W2):
    return pl.pallas_call(
        _mid_body,
        grid=(N // BR,),
        in_specs=[_acc_spec, _row_spec, _deg_spec, _vec_spec, _vec_spec,
                  _vec_spec, _mat_spec],
        out_specs=[_row_spec, _row_spec],
        out_shape=[jax.ShapeDtypeStruct((N, D), _f32)] * 2,
    )(acc2, h, deg2, b, lnw, lnb, W2)


def _post(acc2, h, deg2, b, lnw, lnb):
    return pl.pallas_call(
        _post_body,
        grid=(N // BR,),
        in_specs=[_acc_spec, _row_spec, _deg_spec, _vec_spec, _vec_spec,
                  _vec_spec],
        out_specs=_row_spec,
        out_shape=jax.ShapeDtypeStruct((N, D), _f32),
    )(acc2, h, deg2, b, lnw, lnb)


def kernel(x, edge_index, W1, b1, ln1_w, ln1_b, W2, b2, ln2_w, ln2_b):
    src = edge_index[0].astype(jnp.int32)
    dst = edge_index[1].astype(jnp.int32)
    pad = E_PAD - E
    src_p = jnp.concatenate([src, jnp.zeros((pad,), jnp.int32)])
    dst_p = jnp.concatenate([dst, jnp.full((pad,), N, jnp.int32)])
    srcs = src_p.reshape(NW, NCHUNK, C)
    dsts = dst_p.reshape(NW, NCHUNK, C)

    b1r = b1.reshape(1, D)
    b2r = b2.reshape(1, D)
    ln1w = ln1_w.reshape(1, D)
    ln1b = ln1_b.reshape(1, D)
    ln2w = ln2_w.reshape(1, D)
    ln2b = ln2_b.reshape(1, D)

    deg2 = _deg_kernel(dsts)
    h1, g1 = _pre(x, W1, deg2)
    acc1 = _scatter_kernel(g1, srcs, dsts)
    h2, g2 = _mid(acc1, h1, deg2, b1r, ln1w, ln1b, W2)
    acc2 = _scatter_kernel(g2, srcs, dsts)
    return _post(acc2, h2, deg2, b2r, ln2w, ln2b)


# speedup vs baseline: 8.5951x; 1.0804x over previous
import functools

import jax
import jax.numpy as jnp
from jax import lax
from jax.experimental import pallas as pl
from jax.experimental.pallas import tpu as pltpu
from jax.experimental.pallas import tpu_sc as plsc

N = 10000
D = 128
E = 320000
EPS = 1e-5

NC = 2
NS = 16
NW = NC * NS
C = 128
NCHUNK = 80
EPW = NCHUNK * C
E_PAD = NW * EPW
ACC_ROWS = 10240
RPT = ACC_ROWS // NS
NBUF = 2
IW = 16
NWIN = NCHUNK // IW
GPW = IW // NBUF
NBUF_D = 4
NGRP_D = NCHUNK // NBUF_D

_mesh = plsc.VectorSubcoreMesh(core_axis_name="c", subcore_axis_name="s")


@functools.partial(
    pl.kernel,
    mesh=_mesh,
    out_type=jax.ShapeDtypeStruct((NC, ACC_ROWS, D), jnp.float32),
    scratch_types=[
        pltpu.VMEM((NCHUNK, C), jnp.int32),
        pltpu.VMEM((C, D), jnp.float32),
        pltpu.VMEM_SHARED((ACC_ROWS, D), jnp.float32),
    ]
    + [pltpu.SemaphoreType.DMA] * NBUF_D,
)
def _deg_kernel(*refs):
    dsts_hbm, out_hbm, didx, buf, acc = refs[:5]
    sems = refs[5:5 + NBUF_D]
    cid = lax.axis_index("c")
    sid = lax.axis_index("s")
    wid = cid * NS + sid

    ld = pltpu.async_copy(dsts_hbm.at[wid], didx, sems[0])

    def _fill(val):
        def body(i, _):
            for k in range(D // 16):
                buf[i, pl.ds(k * 16, 16)] = jnp.full((16,), val, jnp.float32)
            return 0
        lax.fori_loop(0, C, body, 0)

    _fill(0.0)
    for t in range(RPT // C):
        pltpu.sync_copy(buf, acc.at[pl.ds(sid * RPT + t * C, C)])
    _fill(1.0)
    ld.wait()
    plsc.subcore_barrier()

    for k in range(NBUF_D):
        pltpu.async_copy(buf, acc.at[didx.at[k]], sems[k], add=True)

    def group(p, _):
        for k in range(NBUF_D):
            pltpu.make_async_copy(buf, acc.at[didx.at[p * NBUF_D + k]],
                                  sems[k]).wait()

            @pl.when(p + 1 < NGRP_D)
            def _issue(k=k):
                pltpu.async_copy(buf, acc.at[didx.at[(p + 1) * NBUF_D + k]],
                                 sems[k], add=True)
        return 0

    lax.fori_loop(0, NGRP_D, group, 0)
    plsc.subcore_barrier()
    pltpu.sync_copy(acc.at[pl.ds(sid * RPT, RPT)],
                    out_hbm.at[cid].at[pl.ds(sid * RPT, RPT)])


@functools.partial(
    pl.kernel,
    mesh=_mesh,
    out_type=jax.ShapeDtypeStruct((NC, ACC_ROWS, D), jnp.float32),
    scratch_types=[
        pltpu.VMEM((2, IW, C), jnp.int32),
        pltpu.VMEM((2, IW, C), jnp.int32),
    ]
    + [pltpu.VMEM((C, D), jnp.float32)] * NBUF
    + [pltpu.SemaphoreType.DMA] * (1 + 2 * NBUF)
    + [pltpu.VMEM_SHARED((ACC_ROWS, D), jnp.float32)],
)
def _scatter_kernel(*refs):
    g_hbm, srcs_hbm, dsts_hbm, out_hbm, swin, dwin = refs[:6]
    bufs = refs[6:6 + NBUF]
    sem_i = refs[6 + NBUF]
    sem_g = refs[7 + NBUF:7 + 2 * NBUF]
    sem_s = refs[7 + 2 * NBUF:7 + 3 * NBUF]
    acc = refs[-1]
    cid = lax.axis_index("c")
    sid = lax.axis_index("s")
    wid = cid * NS + sid
    buf0 = bufs[0]

    def _win_load(w, par):
        pltpu.async_copy(srcs_hbm.at[wid].at[pl.ds(w * IW, IW)],
                         swin.at[par], sem_i)
        pltpu.async_copy(dsts_hbm.at[wid].at[pl.ds(w * IW, IW)],
                         dwin.at[par], sem_i)

    def _win_wait(w, par):
        pltpu.make_async_copy(srcs_hbm.at[wid].at[pl.ds(w * IW, IW)],
                              swin.at[par], sem_i).wait()
        pltpu.make_async_copy(dsts_hbm.at[wid].at[pl.ds(w * IW, IW)],
                              dwin.at[par], sem_i).wait()

    _win_load(0, 0)

    def zero(i, _):
        for k in range(D // 16):
            buf0[i, pl.ds(k * 16, 16)] = jnp.zeros((16,), jnp.float32)
        return 0

    lax.fori_loop(0, C, zero, 0)
    for t in range(RPT // C):
        pltpu.sync_copy(buf0, acc.at[pl.ds(sid * RPT + t * C, C)])
    plsc.subcore_barrier()

    def window(w, _):
        par = lax.rem(w, 2)
        _win_wait(w, par)

        @pl.when(w + 1 < NWIN)
        def _next_win():
            _win_load(w + 1, lax.rem(w + 1, 2))

        for k in range(NBUF):
            pltpu.async_copy(g_hbm.at[swin.at[par, k]], bufs[k], sem_g[k])

        def group(p, _):
            descs = []
            for k in range(NBUF):
                l = p * NBUF + k
                pltpu.make_async_copy(g_hbm.at[swin.at[par, l]], bufs[k],
                                      sem_g[k]).wait()
                descs.append(pltpu.async_copy(bufs[k], acc.at[dwin.at[par, l]],
                                              sem_s[k], add=True))
            for k in range(NBUF):
                descs[k].wait()

                @pl.when(p + 1 < GPW)
                def _issue(k=k):
                    pltpu.async_copy(
                        g_hbm.at[swin.at[par, (p + 1) * NBUF + k]],
                        bufs[k], sem_g[k])
            return 0

        lax.fori_loop(0, GPW, group, 0)
        return 0

    lax.fori_loop(0, NWIN, window, 0)
    plsc.subcore_barrier()
    pltpu.sync_copy(acc.at[pl.ds(sid * RPT, RPT)],
                    out_hbm.at[cid].at[pl.ds(sid * RPT, RPT)])


BR = 400


def _dinv_from(deg_ref):
    deg = deg_ref[0] + deg_ref[1] + 1.0
    return deg, lax.rsqrt(deg)


def _pre_body(x_ref, w_ref, deg_ref, h_ref, g_ref):
    h = jnp.dot(x_ref[...], w_ref[...], preferred_element_type=jnp.float32)
    _, dinv = _dinv_from(deg_ref)
    h_ref[...] = h
    g_ref[...] = h * dinv


def _ln_relu(t, lnw, lnb):
    mu = jnp.mean(t, axis=1, keepdims=True)
    var = jnp.mean((t - mu) ** 2, axis=1, keepdims=True)
    return jnp.maximum((t - mu) * lax.rsqrt(var + EPS) * lnw + lnb, 0.0)


def _mid_body(acc_ref, h_ref, deg_ref, b_ref, lnw_ref, lnb_ref, w2_ref,
              h2_ref, g2_ref):
    deg, dinv = _dinv_from(deg_ref)
    t = (acc_ref[0] + acc_ref[1]) * dinv + h_ref[...] / deg + b_ref[...]
    y = _ln_relu(t, lnw_ref[...], lnb_ref[...])
    h2 = jnp.dot(y, w2_ref[...], preferred_element_type=jnp.float32)
    h2_ref[...] = h2
    g2_ref[...] = h2 * dinv


def _post_body(acc_ref, h_ref, deg_ref, b_ref, lnw_ref, lnb_ref, y_ref):
    deg, dinv = _dinv_from(deg_ref)
    t = (acc_ref[0] + acc_ref[1]) * dinv + h_ref[...] / deg + b_ref[...]
    y_ref[...] = _ln_relu(t, lnw_ref[...], lnb_ref[...])


_row_spec = pl.BlockSpec((BR, D), lambda i: (i, 0))
_mat_spec = pl.BlockSpec((D, D), lambda i: (0, 0))
_vec_spec = pl.BlockSpec((1, D), lambda i: (0, 0))
_deg_spec = pl.BlockSpec((NC, BR, D), lambda i: (0, i, 0))
_acc_spec = pl.BlockSpec((NC, BR, D), lambda i: (0, i, 0))
_f32 = jnp.float32


def _pre(x, W, deg2):
    return pl.pallas_call(
        _pre_body,
        grid=(N // BR,),
        in_specs=[_row_spec, _mat_spec, _deg_spec],
        out_specs=[_row_spec, _row_spec],
        out_shape=[jax.ShapeDtypeStruct((N, D), _f32)] * 2,
    )(x, W, deg2)


def _mid(acc2, h, deg2, b, lnw, lnb, W2):
    return pl.pallas_call(
        _mid_body,
        grid=(N // BR,),
        in_specs=[_acc_spec, _row_spec, _deg_spec, _vec_spec, _vec_spec,
                  _vec_spec, _mat_spec],
        out_specs=[_row_spec, _row_spec],
        out_shape=[jax.ShapeDtypeStruct((N, D), _f32)] * 2,
    )(acc2, h, deg2, b, lnw, lnb, W2)


def _post(acc2, h, deg2, b, lnw, lnb):
    return pl.pallas_call(
        _post_body,
        grid=(N // BR,),
        in_specs=[_acc_spec, _row_spec, _deg_spec, _vec_spec, _vec_spec,
                  _vec_spec],
        out_specs=_row_spec,
        out_shape=jax.ShapeDtypeStruct((N, D), _f32),
    )(acc2, h, deg2, b, lnw, lnb)


def kernel(x, edge_index, W1, b1, ln1_w, ln1_b, W2, b2, ln2_w, ln2_b):
    src = edge_index[0].astype(jnp.int32)
    dst = edge_index[1].astype(jnp.int32)
    pad = E_PAD - E
    src_p = jnp.concatenate([src, jnp.zeros((pad,), jnp.int32)])
    dst_p = jnp.concatenate([dst, jnp.full((pad,), N, jnp.int32)])
    srcs = src_p.reshape(NW, NCHUNK, C)
    dsts = dst_p.reshape(NW, NCHUNK, C)

    b1r = b1.reshape(1, D)
    b2r = b2.reshape(1, D)
    ln1w = ln1_w.reshape(1, D)
    ln1b = ln1_b.reshape(1, D)
    ln2w = ln2_w.reshape(1, D)
    ln2b = ln2_b.reshape(1, D)

    deg2 = _deg_kernel(dsts)
    h1, g1 = _pre(x, W1, deg2)
    acc1 = _scatter_kernel(g1, srcs, dsts)
    h2, g2 = _mid(acc1, h1, deg2, b1r, ln1w, ln1b, W2)
    acc2 = _scatter_kernel(g2, srcs, dsts)
    return _post(acc2, h2, deg2, b2r, ln2w, ln2b)


# speedup vs baseline: 10.3458x; 1.2037x over previous
import functools

import jax
import jax.numpy as jnp
from jax import lax
from jax.experimental import pallas as pl
from jax.experimental.pallas import tpu as pltpu
from jax.experimental.pallas import tpu_sc as plsc

N = 10000
D = 128
E = 320000
EPS = 1e-5

NC = 2
NS = 16
NW = NC * NS
C = 128
A_CHUNK = 128
B_CHUNK = 32
E_PAD = NS * (A_CHUNK + B_CHUNK) * C
ACC_ROWS = 10240
RPT = ACC_ROWS // NS
NBUF = 2
IW = 16
NWIN_A = A_CHUNK // IW
NWIN_B = B_CHUNK // IW
GPW = IW // NBUF
NBUF_D = 4

_mesh = plsc.VectorSubcoreMesh(core_axis_name="c", subcore_axis_name="s")


@functools.partial(
    pl.kernel,
    mesh=_mesh,
    out_type=jax.ShapeDtypeStruct((NC, ACC_ROWS, D), jnp.float32),
    scratch_types=[
        pltpu.VMEM((A_CHUNK, C), jnp.int32),
        pltpu.VMEM((C, D), jnp.float32),
        pltpu.VMEM_SHARED((ACC_ROWS, D), jnp.float32),
    ]
    + [pltpu.SemaphoreType.DMA] * NBUF_D,
)
def _deg_kernel(*refs):
    dsts_a_hbm, dsts_b_hbm, out_hbm, didx, buf, acc = refs[:6]
    sems = refs[6:6 + NBUF_D]
    cid = lax.axis_index("c")
    sid = lax.axis_index("s")

    def _fill(val):
        def body(i, _):
            for k in range(D // 16):
                buf[i, pl.ds(k * 16, 16)] = jnp.full((16,), val, jnp.float32)
            return 0
        lax.fori_loop(0, C, body, 0)

    @pl.when(cid == 0)
    def _ld_a():
        pltpu.async_copy(dsts_a_hbm.at[sid], didx, sems[0])

    @pl.when(cid == 1)
    def _ld_b():
        pltpu.async_copy(dsts_b_hbm.at[sid], didx.at[pl.ds(0, B_CHUNK)],
                         sems[0])

    _fill(0.0)
    for t in range(RPT // C):
        pltpu.sync_copy(buf, acc.at[pl.ds(sid * RPT + t * C, C)])
    _fill(1.0)

    @pl.when(cid == 0)
    def _wt_a():
        pltpu.make_async_copy(dsts_a_hbm.at[sid], didx, sems[0]).wait()

    @pl.when(cid == 1)
    def _wt_b():
        pltpu.make_async_copy(dsts_b_hbm.at[sid], didx.at[pl.ds(0, B_CHUNK)],
                              sems[0]).wait()

    plsc.subcore_barrier()

    def _scan(nchunk):
        ngrp = nchunk // NBUF_D
        for k in range(NBUF_D):
            pltpu.async_copy(buf, acc.at[didx.at[k]], sems[k], add=True)

        def group(p, _):
            for k in range(NBUF_D):
                pltpu.make_async_copy(buf, acc.at[didx.at[p * NBUF_D + k]],
                                      sems[k]).wait()

                @pl.when(p + 1 < ngrp)
                def _issue(k=k):
                    pltpu.async_copy(buf,
                                     acc.at[didx.at[(p + 1) * NBUF_D + k]],
                                     sems[k], add=True)
            return 0

        lax.fori_loop(0, ngrp, group, 0)

    @pl.when(cid == 0)
    def _run_a():
        _scan(A_CHUNK)

    @pl.when(cid == 1)
    def _run_b():
        _scan(B_CHUNK)

    plsc.subcore_barrier()
    pltpu.sync_copy(acc.at[pl.ds(sid * RPT, RPT)],
                    out_hbm.at[cid].at[pl.ds(sid * RPT, RPT)])


@functools.partial(
    pl.kernel,
    mesh=_mesh,
    out_type=jax.ShapeDtypeStruct((NC, ACC_ROWS, D), jnp.float32),
    scratch_types=[
        pltpu.VMEM((2, IW, C), jnp.int32),
        pltpu.VMEM((2, IW, C), jnp.int32),
    ]
    + [pltpu.VMEM((C, D), jnp.float32)] * NBUF
    + [pltpu.SemaphoreType.DMA] * (1 + 2 * NBUF)
    + [pltpu.VMEM_SHARED((ACC_ROWS, D), jnp.float32)],
)
def _scatter_kernel(*refs):
    g_hbm, srcs_a, dsts_a, srcs_b, dsts_b, out_hbm, swin, dwin = refs[:8]
    bufs = refs[8:8 + NBUF]
    sem_i = refs[8 + NBUF]
    sem_g = refs[9 + NBUF:9 + 2 * NBUF]
    sem_s = refs[9 + 2 * NBUF:9 + 3 * NBUF]
    acc = refs[-1]
    cid = lax.axis_index("c")
    sid = lax.axis_index("s")
    buf0 = bufs[0]

    def zero(i, _):
        for k in range(D // 16):
            buf0[i, pl.ds(k * 16, 16)] = jnp.zeros((16,), jnp.float32)
        return 0

    lax.fori_loop(0, C, zero, 0)
    for t in range(RPT // C):
        pltpu.sync_copy(buf0, acc.at[pl.ds(sid * RPT + t * C, C)])
    plsc.subcore_barrier()

    def _run(nwin, srcs_hbm, dsts_hbm):
        def _win_load(w, par):
            pltpu.async_copy(srcs_hbm.at[sid].at[pl.ds(w * IW, IW)],
                             swin.at[par], sem_i)
            pltpu.async_copy(dsts_hbm.at[sid].at[pl.ds(w * IW, IW)],
                             dwin.at[par], sem_i)

        def _win_wait(w, par):
            pltpu.make_async_copy(srcs_hbm.at[sid].at[pl.ds(w * IW, IW)],
                                  swin.at[par], sem_i).wait()
            pltpu.make_async_copy(dsts_hbm.at[sid].at[pl.ds(w * IW, IW)],
                                  dwin.at[par], sem_i).wait()

        _win_load(0, 0)

        def window(w, _):
            par = lax.rem(w, 2)
            _win_wait(w, par)

            @pl.when(w + 1 < nwin)
            def _next_win():
                _win_load(w + 1, lax.rem(w + 1, 2))

            for k in range(NBUF):
                pltpu.async_copy(g_hbm.at[swin.at[par, k]], bufs[k], sem_g[k])

            def group(p, _):
                descs = []
                for k in range(NBUF):
                    l = p * NBUF + k
                    pltpu.make_async_copy(g_hbm.at[swin.at[par, l]], bufs[k],
                                          sem_g[k]).wait()
                    descs.append(
                        pltpu.async_copy(bufs[k], acc.at[dwin.at[par, l]],
                                         sem_s[k], add=True))
                for k in range(NBUF):
                    descs[k].wait()

                    @pl.when(p + 1 < GPW)
                    def _issue(k=k):
                        pltpu.async_copy(
                            g_hbm.at[swin.at[par, (p + 1) * NBUF + k]],
                            bufs[k], sem_g[k])
                return 0

            lax.fori_loop(0, GPW, group, 0)
            return 0

        lax.fori_loop(0, nwin, window, 0)

    @pl.when(cid == 0)
    def _run_a():
        _run(NWIN_A, srcs_a, dsts_a)

    @pl.when(cid == 1)
    def _run_b():
        _run(NWIN_B, srcs_b, dsts_b)

    plsc.subcore_barrier()
    pltpu.sync_copy(acc.at[pl.ds(sid * RPT, RPT)],
                    out_hbm.at[cid].at[pl.ds(sid * RPT, RPT)])


BR = 400


def _dinv_from(deg_ref):
    deg = deg_ref[0] + deg_ref[1] + 1.0
    return deg, lax.rsqrt(deg)


def _pre_body(x_ref, w_ref, deg_ref, h_ref, g_ref):
    h = jnp.dot(x_ref[...], w_ref[...], preferred_element_type=jnp.float32)
    _, dinv = _dinv_from(deg_ref)
    h_ref[...] = h
    g_ref[...] = h * dinv


def _ln_relu(t, lnw, lnb):
    mu = jnp.mean(t, axis=1, keepdims=True)
    var = jnp.mean((t - mu) ** 2, axis=1, keepdims=True)
    return jnp.maximum((t - mu) * lax.rsqrt(var + EPS) * lnw + lnb, 0.0)


def _mid_body(acc_ref, h_ref, deg_ref, b_ref, lnw_ref, lnb_ref, w2_ref,
              h2_ref, g2_ref):
    deg, dinv = _dinv_from(deg_ref)
    t = (acc_ref[0] + acc_ref[1]) * dinv + h_ref[...] / deg + b_ref[...]
    y = _ln_relu(t, lnw_ref[...], lnb_ref[...])
    h2 = jnp.dot(y, w2_ref[...], preferred_element_type=jnp.float32)
    h2_ref[...] = h2
    g2_ref[...] = h2 * dinv


def _post_body(acc_ref, h_ref, deg_ref, b_ref, lnw_ref, lnb_ref, y_ref):
    deg, dinv = _dinv_from(deg_ref)
    t = (acc_ref[0] + acc_ref[1]) * dinv + h_ref[...] / deg + b_ref[...]
    y_ref[...] = _ln_relu(t, lnw_ref[...], lnb_ref[...])


_row_spec = pl.BlockSpec((BR, D), lambda i: (i, 0))
_mat_spec = pl.BlockSpec((D, D), lambda i: (0, 0))
_vec_spec = pl.BlockSpec((1, D), lambda i: (0, 0))
_deg_spec = pl.BlockSpec((NC, BR, D), lambda i: (0, i, 0))
_acc_spec = pl.BlockSpec((NC, BR, D), lambda i: (0, i, 0))
_f32 = jnp.float32


def _pre(x, W, deg2):
    return pl.pallas_call(
        _pre_body,
        grid=(N // BR,),
        in_specs=[_row_spec, _mat_spec, _deg_spec],
        out_specs=[_row_spec, _row_spec],
        out_shape=[jax.ShapeDtypeStruct((N, D), _f32)] * 2,
    )(x, W, deg2)


def _mid(acc2, h, deg2, b, lnw, lnb, W2):
    return pl.pallas_call(
        _mid_body,
        grid=(N // BR,),
        in_specs=[_acc_spec, _row_spec, _deg_spec, _vec_spec, _vec_spec,
                  _vec_spec, _mat_spec],
        out_specs=[_row_spec, _row_spec],
        out_shape=[jax.ShapeDtypeStruct((N, D), _f32)] * 2,
    )(acc2, h, deg2, b, lnw, lnb, W2)


def _post(acc2, h, deg2, b, lnw, lnb):
    return pl.pallas_call(
        _post_body,
        grid=(N // BR,),
        in_specs=[_acc_spec, _row_spec, _deg_spec, _vec_spec, _vec_spec,
                  _vec_spec],
        out_specs=_row_spec,
        out_shape=jax.ShapeDtypeStruct((N, D), _f32),
    )(acc2, h, deg2, b, lnw, lnb)


def kernel(x, edge_index, W1, b1, ln1_w, ln1_b, W2, b2, ln2_w, ln2_b):
    src = edge_index[0].astype(jnp.int32)
    dst = edge_index[1].astype(jnp.int32)
    pad = E_PAD - E
    src_p = jnp.concatenate([src, jnp.zeros((pad,), jnp.int32)])
    dst_p = jnp.concatenate([dst, jnp.full((pad,), N, jnp.int32)])
    n_a = NS * A_CHUNK * C
    srcs_a = src_p[:n_a].reshape(NS, A_CHUNK, C)
    srcs_b = src_p[n_a:].reshape(NS, B_CHUNK, C)
    dsts_a = dst_p[:n_a].reshape(NS, A_CHUNK, C)
    dsts_b = dst_p[n_a:].reshape(NS, B_CHUNK, C)

    b1r = b1.reshape(1, D)
    b2r = b2.reshape(1, D)
    ln1w = ln1_w.reshape(1, D)
    ln1b = ln1_b.reshape(1, D)
    ln2w = ln2_w.reshape(1, D)
    ln2b = ln2_b.reshape(1, D)

    deg2 = _deg_kernel(dsts_a, dsts_b)
    h1, g1 = _pre(x, W1, deg2)
    acc1 = _scatter_kernel(g1, srcs_a, dsts_a, srcs_b, dsts_b)
    h2, g2 = _mid(acc1, h1, deg2, b1r, ln1w, ln1b, W2)
    acc2 = _scatter_kernel(g2, srcs_a, dsts_a, srcs_b, dsts_b)
    return _post(acc2, h2, deg2, b2r, ln2w, ln2b)


# speedup vs baseline: 10.7139x; 1.0356x over previous
import functools

import jax
import jax.numpy as jnp
from jax import lax
from jax.experimental import pallas as pl
from jax.experimental.pallas import tpu as pltpu
from jax.experimental.pallas import tpu_sc as plsc

N = 10000
D = 128
E = 320000
EPS = 1e-5

NC = 2
NS = 16
NW = NC * NS
C = 128
A_CHUNK = 144
B_CHUNK = 16
E_PAD = NS * (A_CHUNK + B_CHUNK) * C
ACC_ROWS = 10240
RPT = ACC_ROWS // NS
NBUF = 2
IW = 16
NWIN_A = A_CHUNK // IW
NWIN_B = B_CHUNK // IW
GPW = IW // NBUF
NBUF_D = 4

_mesh = plsc.VectorSubcoreMesh(core_axis_name="c", subcore_axis_name="s")


@functools.partial(
    pl.kernel,
    mesh=_mesh,
    out_type=jax.ShapeDtypeStruct((NC, ACC_ROWS, D), jnp.float32),
    scratch_types=[
        pltpu.VMEM((A_CHUNK, C), jnp.int32),
        pltpu.VMEM((C, D), jnp.float32),
        pltpu.VMEM_SHARED((ACC_ROWS, D), jnp.float32),
    ]
    + [pltpu.SemaphoreType.DMA] * NBUF_D,
)
def _deg_kernel(*refs):
    dsts_a_hbm, dsts_b_hbm, out_hbm, didx, buf, acc = refs[:6]
    sems = refs[6:6 + NBUF_D]
    cid = lax.axis_index("c")
    sid = lax.axis_index("s")

    def _fill(val):
        def body(i, _):
            for k in range(D // 16):
                buf[i, pl.ds(k * 16, 16)] = jnp.full((16,), val, jnp.float32)
            return 0
        lax.fori_loop(0, C, body, 0)

    @pl.when(cid == 0)
    def _ld_a():
        pltpu.async_copy(dsts_a_hbm.at[sid], didx, sems[0])

    @pl.when(cid == 1)
    def _ld_b():
        pltpu.async_copy(dsts_b_hbm.at[sid], didx.at[pl.ds(0, B_CHUNK)],
                         sems[0])

    _fill(0.0)
    for t in range(RPT // C):
        pltpu.sync_copy(buf, acc.at[pl.ds(sid * RPT + t * C, C)])
    _fill(1.0)

    @pl.when(cid == 0)
    def _wt_a():
        pltpu.make_async_copy(dsts_a_hbm.at[sid], didx, sems[0]).wait()

    @pl.when(cid == 1)
    def _wt_b():
        pltpu.make_async_copy(dsts_b_hbm.at[sid], didx.at[pl.ds(0, B_CHUNK)],
                              sems[0]).wait()

    plsc.subcore_barrier()

    def _scan(nchunk):
        ngrp = nchunk // NBUF_D
        for k in range(NBUF_D):
            pltpu.async_copy(buf, acc.at[didx.at[k]], sems[k], add=True)

        def group(p, _):
            for k in range(NBUF_D):
                pltpu.make_async_copy(buf, acc.at[didx.at[p * NBUF_D + k]],
                                      sems[k]).wait()

                @pl.when(p + 1 < ngrp)
                def _issue(k=k):
                    pltpu.async_copy(buf,
                                     acc.at[didx.at[(p + 1) * NBUF_D + k]],
                                     sems[k], add=True)
            return 0

        lax.fori_loop(0, ngrp, group, 0)

    @pl.when(cid == 0)
    def _run_a():
        _scan(A_CHUNK)

    @pl.when(cid == 1)
    def _run_b():
        _scan(B_CHUNK)

    plsc.subcore_barrier()
    pltpu.sync_copy(acc.at[pl.ds(sid * RPT, RPT)],
                    out_hbm.at[cid].at[pl.ds(sid * RPT, RPT)])


@functools.partial(
    pl.kernel,
    mesh=_mesh,
    out_type=jax.ShapeDtypeStruct((NC, ACC_ROWS, D), jnp.float32),
    scratch_types=[
        pltpu.VMEM((2, IW, C), jnp.int32),
        pltpu.VMEM((2, IW, C), jnp.int32),
    ]
    + [pltpu.VMEM((C, D), jnp.float32)] * NBUF
    + [pltpu.SemaphoreType.DMA] * (1 + 2 * NBUF)
    + [pltpu.VMEM_SHARED((ACC_ROWS, D), jnp.float32)],
)
def _scatter_kernel(*refs):
    g_hbm, srcs_a, dsts_a, srcs_b, dsts_b, out_hbm, swin, dwin = refs[:8]
    bufs = refs[8:8 + NBUF]
    sem_i = refs[8 + NBUF]
    sem_g = refs[9 + NBUF:9 + 2 * NBUF]
    sem_s = refs[9 + 2 * NBUF:9 + 3 * NBUF]
    acc = refs[-1]
    cid = lax.axis_index("c")
    sid = lax.axis_index("s")
    buf0 = bufs[0]

    def zero(i, _):
        for k in range(D // 16):
            buf0[i, pl.ds(k * 16, 16)] = jnp.zeros((16,), jnp.float32)
        return 0

    lax.fori_loop(0, C, zero, 0)
    for t in range(RPT // C):
        pltpu.sync_copy(buf0, acc.at[pl.ds(sid * RPT + t * C, C)])
    plsc.subcore_barrier()

    def _run(nwin, srcs_hbm, dsts_hbm):
        def _win_load(w, par):
            pltpu.async_copy(srcs_hbm.at[sid].at[pl.ds(w * IW, IW)],
                             swin.at[par], sem_i)
            pltpu.async_copy(dsts_hbm.at[sid].at[pl.ds(w * IW, IW)],
                             dwin.at[par], sem_i)

        def _win_wait(w, par):
            pltpu.make_async_copy(srcs_hbm.at[sid].at[pl.ds(w * IW, IW)],
                                  swin.at[par], sem_i).wait()
            pltpu.make_async_copy(dsts_hbm.at[sid].at[pl.ds(w * IW, IW)],
                                  dwin.at[par], sem_i).wait()

        _win_load(0, 0)

        def window(w, _):
            par = lax.rem(w, 2)
            _win_wait(w, par)

            @pl.when(w + 1 < nwin)
            def _next_win():
                _win_load(w + 1, lax.rem(w + 1, 2))

            for k in range(NBUF):
                pltpu.async_copy(g_hbm.at[swin.at[par, k]], bufs[k], sem_g[k])

            def group(p, _):
                descs = []
                for k in range(NBUF):
                    l = p * NBUF + k
                    pltpu.make_async_copy(g_hbm.at[swin.at[par, l]], bufs[k],
                                          sem_g[k]).wait()
                    descs.append(
                        pltpu.async_copy(bufs[k], acc.at[dwin.at[par, l]],
                                         sem_s[k], add=True))
                for k in range(NBUF):
                    descs[k].wait()

                    @pl.when(p + 1 < GPW)
                    def _issue(k=k):
                        pltpu.async_copy(
                            g_hbm.at[swin.at[par, (p + 1) * NBUF + k]],
                            bufs[k], sem_g[k])
                return 0

            lax.fori_loop(0, GPW, group, 0)
            return 0

        lax.fori_loop(0, nwin, window, 0)

    @pl.when(cid == 0)
    def _run_a():
        _run(NWIN_A, srcs_a, dsts_a)

    @pl.when(cid == 1)
    def _run_b():
        _run(NWIN_B, srcs_b, dsts_b)

    plsc.subcore_barrier()
    pltpu.sync_copy(acc.at[pl.ds(sid * RPT, RPT)],
                    out_hbm.at[cid].at[pl.ds(sid * RPT, RPT)])


BR = 400


def _dinv_from(deg_ref):
    deg = deg_ref[0] + deg_ref[1] + 1.0
    return deg, lax.rsqrt(deg)


def _pre_body(x_ref, w_ref, deg_ref, h_ref, g_ref):
    h = jnp.dot(x_ref[...], w_ref[...], preferred_element_type=jnp.float32)
    _, dinv = _dinv_from(deg_ref)
    h_ref[...] = h
    g_ref[...] = h * dinv


def _ln_relu(t, lnw, lnb):
    mu = jnp.mean(t, axis=1, keepdims=True)
    var = jnp.mean((t - mu) ** 2, axis=1, keepdims=True)
    return jnp.maximum((t - mu) * lax.rsqrt(var + EPS) * lnw + lnb, 0.0)


def _mid_body(acc_ref, h_ref, deg_ref, b_ref, lnw_ref, lnb_ref, w2_ref,
              h2_ref, g2_ref):
    deg, dinv = _dinv_from(deg_ref)
    t = (acc_ref[0] + acc_ref[1]) * dinv + h_ref[...] / deg + b_ref[...]
    y = _ln_relu(t, lnw_ref[...], lnb_ref[...])
    h2 = jnp.dot(y, w2_ref[...], preferred_element_type=jnp.float32)
    h2_ref[...] = h2
    g2_ref[...] = h2 * dinv


def _post_body(acc_ref, h_ref, deg_ref, b_ref, lnw_ref, lnb_ref, y_ref):
    deg, dinv = _dinv_from(deg_ref)
    t = (acc_ref[0] + acc_ref[1]) * dinv + h_ref[...] / deg + b_ref[...]
    y_ref[...] = _ln_relu(t, lnw_ref[...], lnb_ref[...])


_row_spec = pl.BlockSpec((BR, D), lambda i: (i, 0))
_mat_spec = pl.BlockSpec((D, D), lambda i: (0, 0))
_vec_spec = pl.BlockSpec((1, D), lambda i: (0, 0))
_deg_spec = pl.BlockSpec((NC, BR, D), lambda i: (0, i, 0))
_acc_spec = pl.BlockSpec((NC, BR, D), lambda i: (0, i, 0))
_f32 = jnp.float32


def _pre(x, W, deg2):
    return pl.pallas_call(
        _pre_body,
        grid=(N // BR,),
        in_specs=[_row_spec, _mat_spec, _deg_spec],
        out_specs=[_row_spec, _row_spec],
        out_shape=[jax.ShapeDtypeStruct((N, D), _f32)] * 2,
    )(x, W, deg2)


def _mid(acc2, h, deg2, b, lnw, lnb, W2):
    return pl.pallas_call(
        _mid_body,
        grid=(N // BR,),
        in_specs=[_acc_spec, _row_spec, _deg_spec, _vec_spec, _vec_spec,
                  _vec_spec, _mat_spec],
        out_specs=[_row_spec, _row_spec],
        out_shape=[jax.ShapeDtypeStruct((N, D), _f32)] * 2,
    )(acc2, h, deg2, b, lnw, lnb, W2)


def _post(acc2, h, deg2, b, lnw, lnb):
    return pl.pallas_call(
        _post_body,
        grid=(N // BR,),
        in_specs=[_acc_spec, _row_spec, _deg_spec, _vec_spec, _vec_spec,
                  _vec_spec],
        out_specs=_row_spec,
        out_shape=jax.ShapeDtypeStruct((N, D), _f32),
    )(acc2, h, deg2, b, lnw, lnb)


def kernel(x, edge_index, W1, b1, ln1_w, ln1_b, W2, b2, ln2_w, ln2_b):
    src = edge_index[0].astype(jnp.int32)
    dst = edge_index[1].astype(jnp.int32)
    pad = E_PAD - E
    src_p = jnp.concatenate([src, jnp.zeros((pad,), jnp.int32)])
    dst_p = jnp.concatenate([dst, jnp.full((pad,), N, jnp.int32)])
    n_a = NS * A_CHUNK * C
    srcs_a = src_p[:n_a].reshape(NS, A_CHUNK, C)
    srcs_b = src_p[n_a:].reshape(NS, B_CHUNK, C)
    dsts_a = dst_p[:n_a].reshape(NS, A_CHUNK, C)
    dsts_b = dst_p[n_a:].reshape(NS, B_CHUNK, C)

    b1r = b1.reshape(1, D)
    b2r = b2.reshape(1, D)
    ln1w = ln1_w.reshape(1, D)
    ln1b = ln1_b.reshape(1, D)
    ln2w = ln2_w.reshape(1, D)
    ln2b = ln2_b.reshape(1, D)

    deg2 = _deg_kernel(dsts_a, dsts_b)
    h1, g1 = _pre(x, W1, deg2)
    acc1 = _scatter_kernel(g1, srcs_a, dsts_a, srcs_b, dsts_b)
    h2, g2 = _mid(acc1, h1, deg2, b1r, ln1w, ln1b, W2)
    acc2 = _scatter_kernel(g2, srcs_a, dsts_a, srcs_b, dsts_b)
    return _post(acc2, h2, deg2, b2r, ln2w, ln2b)
